# baseline (speedup 1.0000x reference)
import functools

import jax
import jax.numpy as jnp
from jax import lax
from jax.experimental import pallas as pl
from jax.experimental.pallas import tpu as pltpu

_FEATURE = 128
_DATA = 262144
_TEMP = 0.07
_BATCH = 128
_CH3 = 3
_IMG = 224

_KBLK = 18816
_NBLK = 16384


def _embed_body(ng, img_ref, w_ref, emb_ref, embT_ref, acc_ref):
    g = pl.program_id(0)

    @pl.when(g == 0)
    def _init():
        acc_ref[...] = jnp.zeros_like(acc_ref)

    acc_ref[...] += lax.dot_general(img_ref[...], w_ref[...],
                                    (((0,), (0,)), ((), ())),
                                    preferred_element_type=jnp.float32)

    @pl.when(g == ng - 1)
    def _finish():
        acc = acc_ref[...]
        norm = jnp.sqrt(jnp.sum(acc * acc, axis=1, keepdims=True)) + 1e-12
        emb = acc / norm
        emb_ref[...] = emb
        embT_ref[...] = emb.T


def _bank_body(emb_ref, embT_ref, idx_ref, bank_ref, logits_ref, nb_ref):
    bank = bank_ref[...]
    logits_ref[...] = jnp.dot(emb_ref[...], bank,
                              preferred_element_type=jnp.float32) * (1.0 / _TEMP)
    j = pl.program_id(0)
    cols = lax.broadcasted_iota(jnp.int32, (_BATCH, _NBLK), 1) + j * _NBLK
    match = (idx_ref[...] == cols).astype(jnp.float32)
    sel = lax.dot_general(embT_ref[...], match,
                          (((1,), (0,)), ((), ())),
                          preferred_element_type=jnp.float32)
    hit = jnp.max(match, axis=0, keepdims=True)
    nb_ref[...] = jnp.where(hit > 0.0, sel, bank)


def kernel(images, W, memory_bank, indices):
    kdim = _CH3 * _IMG * _IMG
    imgT = jnp.transpose(images, (1, 2, 3, 0)).reshape(kdim, _BATCH)
    ng = kdim // _KBLK
    emb, embT = pl.pallas_call(
        functools.partial(_embed_body, ng),
        grid=(ng,),
        in_specs=[
            pl.BlockSpec((_KBLK, _BATCH), lambda g: (g, 0)),
            pl.BlockSpec((_KBLK, _FEATURE), lambda g: (g, 0)),
        ],
        out_specs=[
            pl.BlockSpec((_BATCH, _FEATURE), lambda g: (0, 0)),
            pl.BlockSpec((_FEATURE, _BATCH), lambda g: (0, 0)),
        ],
        out_shape=[
            jax.ShapeDtypeStruct((_BATCH, _FEATURE), jnp.float32),
            jax.ShapeDtypeStruct((_FEATURE, _BATCH), jnp.float32),
        ],
        scratch_shapes=[pltpu.VMEM((_BATCH, _FEATURE), jnp.float32)],
    )(imgT, W)

    ar = jnp.arange(_BATCH)
    dup_later = jnp.any(
        (indices[None, :] == indices[:, None]) & (ar[None, :] > ar[:, None]),
        axis=1)
    scatter_idx = jnp.where(dup_later, -1, indices).reshape(_BATCH, 1)

    nj = _DATA // _NBLK
    logits, new_bank = pl.pallas_call(
        _bank_body,
        grid=(nj,),
        in_specs=[
            pl.BlockSpec((_BATCH, _FEATURE), lambda j: (0, 0)),
            pl.BlockSpec((_FEATURE, _BATCH), lambda j: (0, 0)),
            pl.BlockSpec((_BATCH, 1), lambda j: (0, 0)),
            pl.BlockSpec((_FEATURE, _NBLK), lambda j: (0, j)),
        ],
        out_specs=[
            pl.BlockSpec((_BATCH, _NBLK), lambda j: (0, j)),
            pl.BlockSpec((_FEATURE, _NBLK), lambda j: (0, j)),
        ],
        out_shape=[
            jax.ShapeDtypeStruct((_BATCH, _DATA), jnp.float32),
            jax.ShapeDtypeStruct((_FEATURE, _DATA), jnp.float32),
        ],
    )(emb, embT, scatter_idx, memory_bank)

    return (emb, logits, new_bank)


# speedup vs baseline: 1.0058x; 1.0058x over previous
import functools

import jax
import jax.numpy as jnp
from jax import lax
from jax.experimental import pallas as pl
from jax.experimental.pallas import tpu as pltpu

_FEATURE = 128
_DATA = 262144
_TEMP = 0.07
_BATCH = 128
_CH3 = 3
_IMG = 224

_KBLK = 12544
_NBLK = 16384


def _embed_body(ng, img_ref, w_ref, emb_ref, embT_ref, acc_ref):
    g = pl.program_id(0)

    @pl.when(g == 0)
    def _init():
        acc_ref[...] = jnp.zeros_like(acc_ref)

    acc_ref[...] += lax.dot_general(img_ref[...], w_ref[...],
                                    (((0,), (0,)), ((), ())),
                                    preferred_element_type=jnp.float32)

    @pl.when(g == ng - 1)
    def _finish():
        acc = acc_ref[...]
        norm = jnp.sqrt(jnp.sum(acc * acc, axis=1, keepdims=True)) + 1e-12
        emb = acc / norm
        emb_ref[...] = emb
        embT_ref[...] = emb.T


def _bank_body(emb_ref, embT_ref, idx_ref, bank_ref, logits_ref, nb_ref):
    bank = bank_ref[...]
    logits_ref[...] = jnp.dot(emb_ref[...], bank,
                              preferred_element_type=jnp.float32) * (1.0 / _TEMP)
    j = pl.program_id(0)
    cols = lax.broadcasted_iota(jnp.int32, (_BATCH, _NBLK), 1) + j * _NBLK
    match = (idx_ref[...] == cols).astype(jnp.float32)
    sel = lax.dot_general(embT_ref[...], match,
                          (((1,), (0,)), ((), ())),
                          preferred_element_type=jnp.float32)
    hit = jnp.max(match, axis=0, keepdims=True)
    nb_ref[...] = jnp.where(hit > 0.0, sel, bank)


def kernel(images, W, memory_bank, indices):
    kdim = _CH3 * _IMG * _IMG
    imgT = jnp.transpose(images, (1, 2, 3, 0)).reshape(kdim, _BATCH)
    ng = kdim // _KBLK
    emb, embT = pl.pallas_call(
        functools.partial(_embed_body, ng),
        grid=(ng,),
        in_specs=[
            pl.BlockSpec((_KBLK, _BATCH), lambda g: (g, 0)),
            pl.BlockSpec((_KBLK, _FEATURE), lambda g: (g, 0)),
        ],
        out_specs=[
            pl.BlockSpec((_BATCH, _FEATURE), lambda g: (0, 0)),
            pl.BlockSpec((_FEATURE, _BATCH), lambda g: (0, 0)),
        ],
        out_shape=[
            jax.ShapeDtypeStruct((_BATCH, _FEATURE), jnp.float32),
            jax.ShapeDtypeStruct((_FEATURE, _BATCH), jnp.float32),
        ],
        scratch_shapes=[pltpu.VMEM((_BATCH, _FEATURE), jnp.float32)],
    )(imgT, W)

    ar = jnp.arange(_BATCH)
    dup_later = jnp.any(
        (indices[None, :] == indices[:, None]) & (ar[None, :] > ar[:, None]),
        axis=1)
    scatter_idx = jnp.where(dup_later, -1, indices).reshape(_BATCH, 1)

    nj = _DATA // _NBLK
    logits, new_bank = pl.pallas_call(
        _bank_body,
        grid=(nj,),
        in_specs=[
            pl.BlockSpec((_BATCH, _FEATURE), lambda j: (0, 0)),
            pl.BlockSpec((_FEATURE, _BATCH), lambda j: (0, 0)),
            pl.BlockSpec((_BATCH, 1), lambda j: (0, 0)),
            pl.BlockSpec((_FEATURE, _NBLK), lambda j: (0, j)),
        ],
        out_specs=[
            pl.BlockSpec((_BATCH, _NBLK), lambda j: (0, j)),
            pl.BlockSpec((_FEATURE, _NBLK), lambda j: (0, j)),
        ],
        out_shape=[
            jax.ShapeDtypeStruct((_BATCH, _DATA), jnp.float32),
            jax.ShapeDtypeStruct((_FEATURE, _DATA), jnp.float32),
        ],
    )(emb, embT, scatter_idx, memory_bank)

    return (emb, logits, new_bank)
